# Initial kernel scaffold; baseline (speedup 1.0000x reference)
import jax
import jax.numpy as jnp
from jax.experimental import pallas as pl


def kernel(edge_attr, W0, W1, W2):
    raise NotImplementedError("write your pallas kernel here")


# speedup vs baseline: 1.0039x; 1.0039x over previous
import functools

import jax
import jax.numpy as jnp
from jax import lax
from jax.experimental import pallas as pl
from jax.experimental.pallas import tpu as pltpu
from jax.experimental.pallas import tpu_sc as plsc

N_EDGES = 320000
EMB = 128
NLUT = 60

PREP_ROWS = 2500
PREP_BLK = 100
PREP_GRID = PREP_ROWS // PREP_BLK


def _prep_body(ea_ref, w0_ref, w1_ref, w2_ref, cidx_ref, lut_ref):
    l_idx = lax.broadcasted_iota(jnp.int32, (3 * EMB, EMB), 0)
    e_idx = lax.broadcasted_iota(jnp.int32, (3 * EMB, EMB), 1)
    sel = (
        jnp.where(l_idx == 3 * e_idx, 12.0, 0.0)
        + jnp.where(l_idx == 3 * e_idx + 1, 2.0, 0.0)
        + jnp.where(l_idx == 3 * e_idx + 2, 1.0, 0.0)
    ).astype(jnp.float32)
    ea = ea_ref[...].astype(jnp.float32)
    cidx = jnp.dot(ea, sel, preferred_element_type=jnp.float32)
    cidx_ref[...] = cidx.astype(jnp.int32)

    r0 = lax.broadcasted_iota(jnp.int32, (NLUT, 5), 0)
    c0 = lax.broadcasted_iota(jnp.int32, (NLUT, 5), 1)
    oh0 = (r0 // 12 == c0).astype(jnp.float32)
    r1 = lax.broadcasted_iota(jnp.int32, (NLUT, 6), 0)
    c1 = lax.broadcasted_iota(jnp.int32, (NLUT, 6), 1)
    oh1 = ((r1 // 2) % 6 == c1).astype(jnp.float32)
    r2 = lax.broadcasted_iota(jnp.int32, (NLUT, 2), 0)
    c2 = lax.broadcasted_iota(jnp.int32, (NLUT, 2), 1)
    oh2 = (r2 % 2 == c2).astype(jnp.float32)
    lut_ref[...] = (
        jnp.dot(oh0, w0_ref[...], preferred_element_type=jnp.float32)
        + jnp.dot(oh1, w1_ref[...], preferred_element_type=jnp.float32)
        + jnp.dot(oh2, w2_ref[...], preferred_element_type=jnp.float32)
    )


_prep_call = pl.pallas_call(
    _prep_body,
    out_shape=[
        jax.ShapeDtypeStruct((PREP_ROWS, EMB), jnp.int32),
        jax.ShapeDtypeStruct((NLUT, EMB), jnp.float32),
    ],
)

NW = 32
ROWS_PER_W = N_EDGES // NW
CH = 80
NCH = ROWS_PER_W // CH


@functools.cache
def _get_sc_gather():
    @functools.partial(
        pl.kernel,
        out_type=jax.ShapeDtypeStruct((N_EDGES, EMB), jnp.float32),
        mesh=plsc.VectorSubcoreMesh(core_axis_name="c", subcore_axis_name="s"),
        scratch_types=[
            pltpu.VMEM((NCH, CH), jnp.int32),
            pltpu.VMEM((CH, EMB), jnp.float32),
            pltpu.SemaphoreType.DMA,
        ],
    )
    def _sc_gather(cidx_hbm, lut_hbm, out_hbm, idx_v, rows_v, gsem):
        cid = lax.axis_index("c")
        sid = lax.axis_index("s")
        wid = sid * 2 + cid
        pltpu.sync_copy(cidx_hbm.at[wid], idx_v)

        def body(ci, carry):
            pltpu.async_copy(lut_hbm.at[idx_v.at[ci]], rows_v, gsem).wait()
            pltpu.sync_copy(
                rows_v, out_hbm.at[pl.ds(wid * ROWS_PER_W + ci * CH, CH)]
            )
            return carry

        lax.fori_loop(0, NCH, body, 0)

    return _sc_gather


def kernel(edge_attr, W0, W1, W2):
    ea2 = edge_attr.reshape(PREP_ROWS, 3 * EMB)
    cidx, lut = _prep_call(ea2, W0, W1, W2)
    cidx2 = cidx.reshape(NW, NCH, CH)
    return _get_sc_gather()(cidx2, lut)


# speedup vs baseline: 1.0063x; 1.0024x over previous
import functools

import jax
import jax.numpy as jnp
from jax import lax
from jax.experimental import pallas as pl
from jax.experimental.pallas import tpu as pltpu
from jax.experimental.pallas import tpu_sc as plsc

N_EDGES = 320000
EMB = 128
NLUT = 60

PREP_ROWS = 2500
PREP_BLK = 100
PREP_GRID = PREP_ROWS // PREP_BLK


def _prep_body(ea_ref, w0_ref, w1_ref, w2_ref, cidx_ref, lut_ref):
    l_idx = lax.broadcasted_iota(jnp.int32, (3 * EMB, EMB), 0)
    e_idx = lax.broadcasted_iota(jnp.int32, (3 * EMB, EMB), 1)
    sel = (
        jnp.where(l_idx == 3 * e_idx, 12.0, 0.0)
        + jnp.where(l_idx == 3 * e_idx + 1, 2.0, 0.0)
        + jnp.where(l_idx == 3 * e_idx + 2, 1.0, 0.0)
    ).astype(jnp.float32)
    ea = ea_ref[...].astype(jnp.float32)
    cidx = jnp.dot(ea, sel, preferred_element_type=jnp.float32)
    cidx_ref[...] = cidx.astype(jnp.int32)

    r0 = lax.broadcasted_iota(jnp.int32, (NLUT, 5), 0)
    c0 = lax.broadcasted_iota(jnp.int32, (NLUT, 5), 1)
    oh0 = (r0 // 12 == c0).astype(jnp.float32)
    r1 = lax.broadcasted_iota(jnp.int32, (NLUT, 6), 0)
    c1 = lax.broadcasted_iota(jnp.int32, (NLUT, 6), 1)
    oh1 = ((r1 // 2) % 6 == c1).astype(jnp.float32)
    r2 = lax.broadcasted_iota(jnp.int32, (NLUT, 2), 0)
    c2 = lax.broadcasted_iota(jnp.int32, (NLUT, 2), 1)
    oh2 = (r2 % 2 == c2).astype(jnp.float32)
    lut_ref[...] = (
        jnp.dot(oh0, w0_ref[...], preferred_element_type=jnp.float32)
        + jnp.dot(oh1, w1_ref[...], preferred_element_type=jnp.float32)
        + jnp.dot(oh2, w2_ref[...], preferred_element_type=jnp.float32)
    )


_prep_call = pl.pallas_call(
    _prep_body,
    out_shape=[
        jax.ShapeDtypeStruct((PREP_ROWS, EMB), jnp.int32),
        jax.ShapeDtypeStruct((NLUT, EMB), jnp.float32),
    ],
)

NW = 32
ROWS_PER_W = N_EDGES // NW
CH = 80
NCH = ROWS_PER_W // CH


NBUF = 5
LOOKAHEAD = 3


@functools.cache
def _get_sc_gather():
    @functools.partial(
        pl.kernel,
        out_type=jax.ShapeDtypeStruct((N_EDGES, EMB), jnp.float32),
        mesh=plsc.VectorSubcoreMesh(core_axis_name="c", subcore_axis_name="s"),
        scratch_types=[
            pltpu.VMEM((NCH, CH), jnp.int32),
            [pltpu.VMEM((CH, EMB), jnp.float32)] * NBUF,
            [pltpu.SemaphoreType.DMA] * NBUF,
            [pltpu.SemaphoreType.DMA] * NBUF,
        ],
    )
    def _sc_gather(cidx_hbm, lut_hbm, out_hbm, idx_v, bufs, gsems, osems):
        cid = lax.axis_index("c")
        sid = lax.axis_index("s")
        wid = sid * 2 + cid
        out_base = wid * ROWS_PER_W
        pltpu.sync_copy(cidx_hbm.at[wid], idx_v)

        def start_gather(ci, k):
            pltpu.async_copy(lut_hbm.at[idx_v.at[ci]], bufs[k], gsems[k])

        def wait_gather(ci, k):
            pltpu.make_async_copy(
                lut_hbm.at[idx_v.at[ci]], bufs[k], gsems[k]
            ).wait()

        def start_out(ci, k):
            pltpu.async_copy(
                bufs[k], out_hbm.at[pl.ds(out_base + ci * CH, CH)], osems[k]
            )

        def wait_out(ci, k):
            pltpu.make_async_copy(
                bufs[k], out_hbm.at[pl.ds(out_base + ci * CH, CH)], osems[k]
            ).wait()

        for b in range(LOOKAHEAD):
            start_gather(b, b)

        def body(g, carry):
            for b in range(NBUF):
                c = g * NBUF + b
                x = (b + LOOKAHEAD) % NBUF
                cg = c + LOOKAHEAD
                co = cg - NBUF

                @pl.when((c >= NBUF - LOOKAHEAD) & (cg < NCH))
                def _():
                    wait_out(co, x)

                @pl.when(cg < NCH)
                def _():
                    start_gather(cg, x)

                wait_gather(c, b)
                start_out(c, b)
            return carry

        lax.fori_loop(0, NCH // NBUF, body, 0)
        for b in range(NBUF):
            wait_out(NCH - NBUF + b, b)

    return _sc_gather


def kernel(edge_attr, W0, W1, W2):
    ea2 = edge_attr.reshape(PREP_ROWS, 3 * EMB)
    cidx, lut = _prep_call(ea2, W0, W1, W2)
    cidx2 = cidx.reshape(NW, NCH, CH)
    return _get_sc_gather()(cidx2, lut)


# speedup vs baseline: 1.2199x; 1.2123x over previous
import functools

import jax
import jax.numpy as jnp
from jax import lax
from jax.experimental import pallas as pl
from jax.experimental.pallas import tpu as pltpu
from jax.experimental.pallas import tpu_sc as plsc

N_EDGES = 320000
EMB = 128
NLUT = 60

PREP_ROWS = 2500
PREP_BLK = 100
PREP_GRID = PREP_ROWS // PREP_BLK


def _prep_body(ea_ref, w0_ref, w1_ref, w2_ref, cidx_ref, lut_ref):
    l_idx = lax.broadcasted_iota(jnp.int32, (3 * EMB, EMB), 0)
    e_idx = lax.broadcasted_iota(jnp.int32, (3 * EMB, EMB), 1)
    sel = (
        jnp.where(l_idx == 3 * e_idx, 12.0, 0.0)
        + jnp.where(l_idx == 3 * e_idx + 1, 2.0, 0.0)
        + jnp.where(l_idx == 3 * e_idx + 2, 1.0, 0.0)
    ).astype(jnp.float32)
    ea = ea_ref[...].astype(jnp.float32)
    cidx = jnp.dot(ea, sel, preferred_element_type=jnp.float32)
    cidx_ref[...] = cidx.astype(jnp.int32)

    r0 = lax.broadcasted_iota(jnp.int32, (NLUT, 5), 0)
    c0 = lax.broadcasted_iota(jnp.int32, (NLUT, 5), 1)
    oh0 = (r0 // 12 == c0).astype(jnp.float32)
    r1 = lax.broadcasted_iota(jnp.int32, (NLUT, 6), 0)
    c1 = lax.broadcasted_iota(jnp.int32, (NLUT, 6), 1)
    oh1 = ((r1 // 2) % 6 == c1).astype(jnp.float32)
    r2 = lax.broadcasted_iota(jnp.int32, (NLUT, 2), 0)
    c2 = lax.broadcasted_iota(jnp.int32, (NLUT, 2), 1)
    oh2 = (r2 % 2 == c2).astype(jnp.float32)
    lut_ref[...] = (
        jnp.dot(oh0, w0_ref[...], preferred_element_type=jnp.float32)
        + jnp.dot(oh1, w1_ref[...], preferred_element_type=jnp.float32)
        + jnp.dot(oh2, w2_ref[...], preferred_element_type=jnp.float32)
    )


_prep_call = pl.pallas_call(
    _prep_body,
    out_shape=[
        jax.ShapeDtypeStruct((PREP_ROWS, EMB), jnp.int32),
        jax.ShapeDtypeStruct((NLUT, EMB), jnp.float32),
    ],
)

NW = 32
ROWS_PER_W = N_EDGES // NW
CH = 80
NCH = ROWS_PER_W // CH


SLAB = 400
NSLAB = ROWS_PER_W // SLAB
GPS = SLAB // 16


SLAB_W = SLAB * EMB


@functools.cache
def _get_sc_gather():
    @functools.partial(
        pl.kernel,
        out_type=jax.ShapeDtypeStruct((N_EDGES * EMB,), jnp.float32),
        mesh=plsc.VectorSubcoreMesh(core_axis_name="c", subcore_axis_name="s"),
        compiler_params=pltpu.CompilerParams(needs_layout_passes=False),
        scratch_types=[
            pltpu.VMEM((ROWS_PER_W,), jnp.int32),
            pltpu.VMEM((NLUT * EMB,), jnp.float32),
            pltpu.VMEM((SLAB_W,), jnp.float32),
            pltpu.VMEM((SLAB_W,), jnp.float32),
            pltpu.SemaphoreType.DMA,
            pltpu.SemaphoreType.DMA,
        ],
    )
    def _sc_gather(cidx_hbm, lut_hbm, out_hbm, idx_v, lut_v, buf0, buf1, o0, o1):
        cid = lax.axis_index("c")
        sid = lax.axis_index("s")
        wid = sid * 2 + cid
        out_base = wid * ROWS_PER_W * EMB
        pltpu.sync_copy(cidx_hbm.at[pl.ds(wid * ROWS_PER_W, ROWS_PER_W)], idx_v)
        pltpu.sync_copy(lut_hbm, lut_v)
        iota16 = lax.broadcasted_iota(jnp.int32, (16,), 0)
        iota128 = iota16 * EMB

        def fill(s, buf):
            def grp(g, carry):
                cvec = idx_v[pl.ds((s * GPS + g) * 16, 16)]
                src = cvec * EMB
                dst = iota128 + g * (16 * EMB)
                for col in range(EMB):
                    vals = plsc.load_gather(lut_v, [src + col])
                    plsc.store_scatter(buf, [dst + col], vals)
                return carry

            lax.fori_loop(0, GPS, grp, 0)

        def start_out(s, buf, sem):
            pltpu.async_copy(
                buf, out_hbm.at[pl.ds(out_base + s * SLAB_W, SLAB_W)], sem
            )

        def wait_out(s, buf, sem):
            pltpu.make_async_copy(
                buf, out_hbm.at[pl.ds(out_base + s * SLAB_W, SLAB_W)], sem
            ).wait()

        fill(0, buf0)
        start_out(0, buf0, o0)
        fill(1, buf1)
        start_out(1, buf1, o1)

        def pair(p, carry):
            s0 = 2 * p + 2
            wait_out(s0 - 2, buf0, o0)
            fill(s0, buf0)
            start_out(s0, buf0, o0)
            s1 = s0 + 1
            wait_out(s1 - 2, buf1, o1)
            fill(s1, buf1)
            start_out(s1, buf1, o1)
            return carry

        lax.fori_loop(0, (NSLAB - 3) // 2, pair, 0)
        wait_out(NSLAB - 3, buf0, o0)
        fill(NSLAB - 1, buf0)
        start_out(NSLAB - 1, buf0, o0)
        wait_out(NSLAB - 2, buf1, o1)
        wait_out(NSLAB - 1, buf0, o0)

    return _sc_gather


def kernel(edge_attr, W0, W1, W2):
    ea2 = edge_attr.reshape(PREP_ROWS, 3 * EMB)
    cidx, lut = _prep_call(ea2, W0, W1, W2)
    out = _get_sc_gather()(cidx.reshape(-1), lut.reshape(-1))
    return out.reshape(N_EDGES, EMB)


# speedup vs baseline: 3.1110x; 2.5502x over previous
import functools

import jax
import jax.numpy as jnp
from jax import lax
from jax.experimental import pallas as pl
from jax.experimental.pallas import tpu as pltpu
from jax.experimental.pallas import tpu_sc as plsc

N_EDGES = 320000
EMB = 128
NLUT = 60

PREP_ROWS = 2500
PREP_BLK = 100
PREP_GRID = PREP_ROWS // PREP_BLK


def _prep_body(ea_ref, w0_ref, w1_ref, w2_ref, cidx_ref, lut_ref):
    l_idx = lax.broadcasted_iota(jnp.int32, (3 * EMB, EMB), 0)
    e_idx = lax.broadcasted_iota(jnp.int32, (3 * EMB, EMB), 1)
    sel = (
        jnp.where(l_idx == 3 * e_idx, 12.0, 0.0)
        + jnp.where(l_idx == 3 * e_idx + 1, 2.0, 0.0)
        + jnp.where(l_idx == 3 * e_idx + 2, 1.0, 0.0)
    ).astype(jnp.float32)
    ea = ea_ref[...].astype(jnp.float32)
    cidx = jnp.dot(ea, sel, preferred_element_type=jnp.float32)
    cidx_ref[...] = cidx.astype(jnp.int32)

    r0 = lax.broadcasted_iota(jnp.int32, (NLUT, 5), 0)
    c0 = lax.broadcasted_iota(jnp.int32, (NLUT, 5), 1)
    oh0 = (r0 // 12 == c0).astype(jnp.float32)
    r1 = lax.broadcasted_iota(jnp.int32, (NLUT, 6), 0)
    c1 = lax.broadcasted_iota(jnp.int32, (NLUT, 6), 1)
    oh1 = ((r1 // 2) % 6 == c1).astype(jnp.float32)
    r2 = lax.broadcasted_iota(jnp.int32, (NLUT, 2), 0)
    c2 = lax.broadcasted_iota(jnp.int32, (NLUT, 2), 1)
    oh2 = (r2 % 2 == c2).astype(jnp.float32)
    lut_ref[...] = (
        jnp.dot(oh0, w0_ref[...], preferred_element_type=jnp.float32)
        + jnp.dot(oh1, w1_ref[...], preferred_element_type=jnp.float32)
        + jnp.dot(oh2, w2_ref[...], preferred_element_type=jnp.float32)
    )


_prep_call = pl.pallas_call(
    _prep_body,
    out_shape=[
        jax.ShapeDtypeStruct((PREP_ROWS, EMB), jnp.int32),
        jax.ShapeDtypeStruct((NLUT, EMB), jnp.float32),
    ],
)

NW = 32
ROWS_PER_W = N_EDGES // NW
CH = 80
NCH = ROWS_PER_W // CH


SLAB = 400
NSLAB = ROWS_PER_W // SLAB
GPS = SLAB // 16


SLAB_W = SLAB * EMB


@functools.cache
def _get_sc_gather():
    @functools.partial(
        pl.kernel,
        out_type=jax.ShapeDtypeStruct((N_EDGES * EMB,), jnp.float32),
        mesh=plsc.VectorSubcoreMesh(core_axis_name="c", subcore_axis_name="s"),
        compiler_params=pltpu.CompilerParams(needs_layout_passes=False),
        scratch_types=[
            pltpu.VMEM((ROWS_PER_W,), jnp.int32),
            pltpu.VMEM((NLUT * EMB,), jnp.float32),
            pltpu.VMEM((SLAB_W,), jnp.float32),
            pltpu.VMEM((SLAB_W,), jnp.float32),
            pltpu.SemaphoreType.DMA,
            pltpu.SemaphoreType.DMA,
        ],
    )
    def _sc_gather(cidx_hbm, lut_hbm, out_hbm, idx_v, lut_v, buf0, buf1, o0, o1):
        cid = lax.axis_index("c")
        sid = lax.axis_index("s")
        wid = sid * 2 + cid
        out_base = wid * ROWS_PER_W * EMB
        pltpu.sync_copy(cidx_hbm.at[pl.ds(wid * ROWS_PER_W, ROWS_PER_W)], idx_v)
        pltpu.sync_copy(lut_hbm, lut_v)
        iota16 = lax.broadcasted_iota(jnp.int32, (16,), 0)
        iota128 = iota16 * EMB

        def fill(s, buf):
            def grp(g, carry):
                cvec = idx_v[pl.ds((s * GPS + g) * 16, 16)]
                src = cvec * EMB
                dst = iota128 + g * (16 * EMB)
                for col in range(EMB):
                    rot = (iota16 + col) & (EMB - 1)
                    vals = plsc.load_gather(lut_v, [src + rot])
                    plsc.store_scatter(buf, [dst + rot], vals)
                return carry

            lax.fori_loop(0, GPS, grp, 0)

        def start_out(s, buf, sem):
            pltpu.async_copy(
                buf, out_hbm.at[pl.ds(out_base + s * SLAB_W, SLAB_W)], sem
            )

        def wait_out(s, buf, sem):
            pltpu.make_async_copy(
                buf, out_hbm.at[pl.ds(out_base + s * SLAB_W, SLAB_W)], sem
            ).wait()

        fill(0, buf0)
        start_out(0, buf0, o0)
        fill(1, buf1)
        start_out(1, buf1, o1)

        def pair(p, carry):
            s0 = 2 * p + 2
            wait_out(s0 - 2, buf0, o0)
            fill(s0, buf0)
            start_out(s0, buf0, o0)
            s1 = s0 + 1
            wait_out(s1 - 2, buf1, o1)
            fill(s1, buf1)
            start_out(s1, buf1, o1)
            return carry

        lax.fori_loop(0, (NSLAB - 3) // 2, pair, 0)
        wait_out(NSLAB - 3, buf0, o0)
        fill(NSLAB - 1, buf0)
        start_out(NSLAB - 1, buf0, o0)
        wait_out(NSLAB - 2, buf1, o1)
        wait_out(NSLAB - 1, buf0, o0)

    return _sc_gather


def kernel(edge_attr, W0, W1, W2):
    ea2 = edge_attr.reshape(PREP_ROWS, 3 * EMB)
    cidx, lut = _prep_call(ea2, W0, W1, W2)
    out = _get_sc_gather()(cidx.reshape(-1), lut.reshape(-1))
    return out.reshape(N_EDGES, EMB)


# speedup vs baseline: 4.1877x; 1.3461x over previous
import functools

import jax
import jax.numpy as jnp
from jax import lax
from jax.experimental import pallas as pl
from jax.experimental.pallas import tpu as pltpu
from jax.experimental.pallas import tpu_sc as plsc

N_EDGES = 320000
EMB = 128
NLUT = 60

PREP_ROWS = 2500
PREP_BLK = 100
PREP_GRID = PREP_ROWS // PREP_BLK


def _prep_body(ea_ref, w0_ref, w1_ref, w2_ref, cidx_ref, lut_ref):
    l_idx = lax.broadcasted_iota(jnp.int32, (3 * EMB, EMB), 0)
    e_idx = lax.broadcasted_iota(jnp.int32, (3 * EMB, EMB), 1)
    sel = (
        jnp.where(l_idx == 3 * e_idx, 12.0, 0.0)
        + jnp.where(l_idx == 3 * e_idx + 1, 2.0, 0.0)
        + jnp.where(l_idx == 3 * e_idx + 2, 1.0, 0.0)
    ).astype(jnp.float32)
    ea = ea_ref[...].astype(jnp.float32)
    cidx = jnp.dot(ea, sel, preferred_element_type=jnp.float32)
    cidx_ref[...] = cidx.astype(jnp.int32)

    r0 = lax.broadcasted_iota(jnp.int32, (NLUT, 5), 0)
    c0 = lax.broadcasted_iota(jnp.int32, (NLUT, 5), 1)
    oh0 = (r0 // 12 == c0).astype(jnp.float32)
    r1 = lax.broadcasted_iota(jnp.int32, (NLUT, 6), 0)
    c1 = lax.broadcasted_iota(jnp.int32, (NLUT, 6), 1)
    oh1 = ((r1 // 2) % 6 == c1).astype(jnp.float32)
    r2 = lax.broadcasted_iota(jnp.int32, (NLUT, 2), 0)
    c2 = lax.broadcasted_iota(jnp.int32, (NLUT, 2), 1)
    oh2 = (r2 % 2 == c2).astype(jnp.float32)
    lut_ref[...] = (
        jnp.dot(oh0, w0_ref[...], preferred_element_type=jnp.float32)
        + jnp.dot(oh1, w1_ref[...], preferred_element_type=jnp.float32)
        + jnp.dot(oh2, w2_ref[...], preferred_element_type=jnp.float32)
    )


_prep_call = pl.pallas_call(
    _prep_body,
    out_shape=[
        jax.ShapeDtypeStruct((PREP_ROWS, EMB), jnp.int32),
        jax.ShapeDtypeStruct((NLUT, EMB), jnp.float32),
    ],
)

NW = 32
ROWS_PER_W = N_EDGES // NW
CH = 80
NCH = ROWS_PER_W // CH


SLAB = 400
NSLAB = ROWS_PER_W // SLAB
GPS = SLAB // 16


SLAB_W = SLAB * EMB


@functools.cache
def _get_sc_gather():
    @functools.partial(
        pl.kernel,
        out_type=jax.ShapeDtypeStruct((N_EDGES * EMB,), jnp.float32),
        mesh=plsc.VectorSubcoreMesh(core_axis_name="c", subcore_axis_name="s"),
        compiler_params=pltpu.CompilerParams(needs_layout_passes=False),
        scratch_types=[
            pltpu.VMEM_SHARED((16 * ROWS_PER_W,), jnp.int32),
            pltpu.VMEM((ROWS_PER_W,), jnp.int32),
            pltpu.VMEM((NLUT * EMB,), jnp.float32),
            pltpu.VMEM((SLAB_W,), jnp.float32),
            pltpu.VMEM((SLAB_W,), jnp.float32),
            pltpu.SMEM((SLAB,), jnp.int32),
            pltpu.SemaphoreType.DMA,
            pltpu.SemaphoreType.DMA,
        ],
    )
    def _sc_gather(
        cidx_hbm, lut_hbm, out_hbm, idx_v, idx_t, lut_v, buf0, buf1, idx_s, o0, o1
    ):
        cid = lax.axis_index("c")
        sid = lax.axis_index("s")
        wid = sid * 2 + cid
        out_base = wid * ROWS_PER_W * EMB
        pltpu.sync_copy(cidx_hbm.at[pl.ds(wid * ROWS_PER_W, ROWS_PER_W)], idx_t)
        pltpu.sync_copy(idx_t, idx_v.at[pl.ds(sid * ROWS_PER_W, ROWS_PER_W)])
        pltpu.sync_copy(lut_hbm, lut_v)

        def fill(s, buf):
            pltpu.sync_copy(
                idx_v.at[pl.ds(sid * ROWS_PER_W + s * SLAB, SLAB)], idx_s
            )

            def edge(e, carry):
                base = idx_s[e] * EMB
                ebase = e * EMB
                for d in range(EMB // 16):
                    buf[pl.ds(ebase + d * 16, 16)] = lut_v[
                        pl.ds(base + d * 16, 16)
                    ]
                return carry

            lax.fori_loop(0, SLAB, edge, 0)

        def start_out(s, buf, sem):
            pltpu.async_copy(
                buf, out_hbm.at[pl.ds(out_base + s * SLAB_W, SLAB_W)], sem
            )

        def wait_out(s, buf, sem):
            pltpu.make_async_copy(
                buf, out_hbm.at[pl.ds(out_base + s * SLAB_W, SLAB_W)], sem
            ).wait()

        fill(0, buf0)
        start_out(0, buf0, o0)
        fill(1, buf1)
        start_out(1, buf1, o1)

        def pair(p, carry):
            s0 = 2 * p + 2
            wait_out(s0 - 2, buf0, o0)
            fill(s0, buf0)
            start_out(s0, buf0, o0)
            s1 = s0 + 1
            wait_out(s1 - 2, buf1, o1)
            fill(s1, buf1)
            start_out(s1, buf1, o1)
            return carry

        lax.fori_loop(0, (NSLAB - 3) // 2, pair, 0)
        wait_out(NSLAB - 3, buf0, o0)
        fill(NSLAB - 1, buf0)
        start_out(NSLAB - 1, buf0, o0)
        wait_out(NSLAB - 2, buf1, o1)
        wait_out(NSLAB - 1, buf0, o0)

    return _sc_gather


def kernel(edge_attr, W0, W1, W2):
    ea2 = edge_attr.reshape(PREP_ROWS, 3 * EMB)
    cidx, lut = _prep_call(ea2, W0, W1, W2)
    out = _get_sc_gather()(cidx.reshape(-1), lut.reshape(-1))
    return out.reshape(N_EDGES, EMB)


# speedup vs baseline: 4.2695x; 1.0195x over previous
import functools

import jax
import jax.numpy as jnp
from jax import lax
from jax.experimental import pallas as pl
from jax.experimental.pallas import tpu as pltpu
from jax.experimental.pallas import tpu_sc as plsc

N_EDGES = 320000
EMB = 128
NLUT = 60

PREP_ROWS = 2500
PREP_BLK = 100
PREP_GRID = PREP_ROWS // PREP_BLK


PREP_BLK_E = 6400
PREP_GRID_N = N_EDGES // PREP_BLK_E


def _prep_body(ea_ref, w0_ref, w1_ref, w2_ref, cidx_ref, lut_ref):
    i = pl.program_id(0)
    ea = ea_ref[...]
    wcol = lax.broadcasted_iota(jnp.int32, (1, 3), 1)
    w = jnp.where(wcol == 0, 12, jnp.where(wcol == 1, 2, 1))
    c = jnp.sum(ea * w, axis=1)
    rows = PREP_BLK_E // EMB
    cidx_ref[pl.ds(i * rows, rows), :] = c.reshape(rows, EMB)

    @pl.when(i == 0)
    def _():
        r0 = lax.broadcasted_iota(jnp.int32, (NLUT, 5), 0)
        c0 = lax.broadcasted_iota(jnp.int32, (NLUT, 5), 1)
        oh0 = (r0 // 12 == c0).astype(jnp.float32)
        r1 = lax.broadcasted_iota(jnp.int32, (NLUT, 6), 0)
        c1 = lax.broadcasted_iota(jnp.int32, (NLUT, 6), 1)
        oh1 = ((r1 // 2) % 6 == c1).astype(jnp.float32)
        r2 = lax.broadcasted_iota(jnp.int32, (NLUT, 2), 0)
        c2 = lax.broadcasted_iota(jnp.int32, (NLUT, 2), 1)
        oh2 = (r2 % 2 == c2).astype(jnp.float32)
        lut_ref[...] = (
            jnp.dot(oh0, w0_ref[...], preferred_element_type=jnp.float32)
            + jnp.dot(oh1, w1_ref[...], preferred_element_type=jnp.float32)
            + jnp.dot(oh2, w2_ref[...], preferred_element_type=jnp.float32)
        )


_prep_call = pl.pallas_call(
    _prep_body,
    grid=(PREP_GRID_N,),
    in_specs=[
        pl.BlockSpec((PREP_BLK_E, 3), lambda i: (i, 0)),
        pl.BlockSpec((5, EMB), lambda i: (0, 0)),
        pl.BlockSpec((6, EMB), lambda i: (0, 0)),
        pl.BlockSpec((2, EMB), lambda i: (0, 0)),
    ],
    out_specs=[
        pl.BlockSpec((PREP_ROWS, EMB), lambda i: (0, 0)),
        pl.BlockSpec((NLUT, EMB), lambda i: (0, 0)),
    ],
    out_shape=[
        jax.ShapeDtypeStruct((PREP_ROWS, EMB), jnp.int32),
        jax.ShapeDtypeStruct((NLUT, EMB), jnp.float32),
    ],
)

NW = 32
ROWS_PER_W = N_EDGES // NW
CH = 80
NCH = ROWS_PER_W // CH


SLAB = 400
NSLAB = ROWS_PER_W // SLAB
GPS = SLAB // 16


SLAB_W = SLAB * EMB


@functools.cache
def _get_sc_gather():
    @functools.partial(
        pl.kernel,
        out_type=jax.ShapeDtypeStruct((N_EDGES * EMB,), jnp.float32),
        mesh=plsc.VectorSubcoreMesh(core_axis_name="c", subcore_axis_name="s"),
        compiler_params=pltpu.CompilerParams(needs_layout_passes=False),
        scratch_types=[
            pltpu.VMEM_SHARED((16 * ROWS_PER_W,), jnp.int32),
            pltpu.VMEM((ROWS_PER_W,), jnp.int32),
            pltpu.VMEM((NLUT * EMB,), jnp.float32),
            pltpu.VMEM((SLAB_W,), jnp.float32),
            pltpu.VMEM((SLAB_W,), jnp.float32),
            pltpu.SMEM((SLAB,), jnp.int32),
            pltpu.SemaphoreType.DMA,
            pltpu.SemaphoreType.DMA,
        ],
    )
    def _sc_gather(
        cidx_hbm, lut_hbm, out_hbm, idx_v, idx_t, lut_v, buf0, buf1, idx_s, o0, o1
    ):
        cid = lax.axis_index("c")
        sid = lax.axis_index("s")
        wid = sid * 2 + cid
        out_base = wid * ROWS_PER_W * EMB
        pltpu.sync_copy(cidx_hbm.at[pl.ds(wid * ROWS_PER_W, ROWS_PER_W)], idx_t)
        pltpu.sync_copy(idx_t, idx_v.at[pl.ds(sid * ROWS_PER_W, ROWS_PER_W)])
        pltpu.sync_copy(lut_hbm, lut_v)

        def fill(s, buf):
            pltpu.sync_copy(
                idx_v.at[pl.ds(sid * ROWS_PER_W + s * SLAB, SLAB)], idx_s
            )

            def edge(e, carry):
                base = idx_s[e] * EMB
                ebase = e * EMB
                for d in range(EMB // 16):
                    buf[pl.ds(ebase + d * 16, 16)] = lut_v[
                        pl.ds(base + d * 16, 16)
                    ]
                return carry

            lax.fori_loop(0, SLAB, edge, 0)

        def start_out(s, buf, sem):
            pltpu.async_copy(
                buf, out_hbm.at[pl.ds(out_base + s * SLAB_W, SLAB_W)], sem
            )

        def wait_out(s, buf, sem):
            pltpu.make_async_copy(
                buf, out_hbm.at[pl.ds(out_base + s * SLAB_W, SLAB_W)], sem
            ).wait()

        fill(0, buf0)
        start_out(0, buf0, o0)
        fill(1, buf1)
        start_out(1, buf1, o1)

        def pair(p, carry):
            s0 = 2 * p + 2
            wait_out(s0 - 2, buf0, o0)
            fill(s0, buf0)
            start_out(s0, buf0, o0)
            s1 = s0 + 1
            wait_out(s1 - 2, buf1, o1)
            fill(s1, buf1)
            start_out(s1, buf1, o1)
            return carry

        lax.fori_loop(0, (NSLAB - 3) // 2, pair, 0)
        wait_out(NSLAB - 3, buf0, o0)
        fill(NSLAB - 1, buf0)
        start_out(NSLAB - 1, buf0, o0)
        wait_out(NSLAB - 2, buf1, o1)
        wait_out(NSLAB - 1, buf0, o0)

    return _sc_gather


def kernel(edge_attr, W0, W1, W2):
    cidx, lut = _prep_call(edge_attr, W0, W1, W2)
    out = _get_sc_gather()(cidx.reshape(-1), lut.reshape(-1))
    return out.reshape(N_EDGES, EMB)
